# Initial kernel scaffold; baseline (speedup 1.0000x reference)
import jax
import jax.numpy as jnp
from jax.experimental import pallas as pl


def kernel(x, edge_index, Wk1, bk1, Wq1, bq1, Wv1, bv1, Ws1, b1, Wk2, bk2, Wq2, bq2, Wv2, bv2, Ws2, b2):
    raise NotImplementedError("write your pallas kernel here")


# speedup vs baseline: 4.8079x; 4.8079x over previous
import functools

import jax
import jax.numpy as jnp
from jax import lax
from jax.experimental import pallas as pl
from jax.experimental.pallas import tpu as pltpu
from jax.experimental.pallas import tpu_sc as plsc

N = 10000
E = 320000
D = 128

NC = 2
NS = 16
NW = NC * NS
EPW = E // NW
C = 80
NCHUNK = EPW // C
NPAD = 10240
RPT = NPAD // NS
RZ = 128

MBLK = 1000
GRID = N // MBLK


def _proj_body(x_ref, w_ref, b_ref, kn_ref, qn_ref, v_ref, s_ref):
    y = jnp.dot(x_ref[...], w_ref[...], preferred_element_type=jnp.float32)
    y = y + b_ref[0:1, :]
    kn_ref[...] = y[:, 0 * D:1 * D]
    qn_ref[...] = y[:, 1 * D:2 * D]
    v_ref[...] = y[:, 2 * D:3 * D]
    s_ref[...] = y[:, 3 * D:4 * D]


def _relu_proj_body(p0_ref, p1_ref, s1_ref, w_ref, b_ref,
                    kn_ref, qn_ref, v_ref, s_ref):
    h = jnp.maximum(p0_ref[...] + p1_ref[...] + s1_ref[...], 0.0)
    y = jnp.dot(h, w_ref[...], preferred_element_type=jnp.float32)
    y = y + b_ref[0:1, :]
    kn_ref[...] = y[:, 0 * D:1 * D]
    qn_ref[...] = y[:, 1 * D:2 * D]
    v_ref[...] = y[:, 2 * D:3 * D]
    s_ref[...] = y[:, 3 * D:4 * D]


def _final_body(p0_ref, p1_ref, s2_ref, o_ref):
    o_ref[...] = p0_ref[...] + p1_ref[...] + s2_ref[...]


_row_spec = pl.BlockSpec((MBLK, D), lambda i: (i, 0))
_w_spec = pl.BlockSpec((D, 4 * D), lambda i: (0, 0))
_b_spec = pl.BlockSpec((8, 4 * D), lambda i: (0, 0))
_out4 = tuple(jax.ShapeDtypeStruct((N, D), jnp.float32) for _ in range(4))

_proj = pl.pallas_call(
    _proj_body,
    grid=(GRID,),
    in_specs=[_row_spec, _w_spec, _b_spec],
    out_specs=(_row_spec,) * 4,
    out_shape=_out4,
)

_relu_proj = pl.pallas_call(
    _relu_proj_body,
    grid=(GRID,),
    in_specs=[_row_spec, _row_spec, _row_spec, _w_spec, _b_spec],
    out_specs=(_row_spec,) * 4,
    out_shape=_out4,
)

_final = pl.pallas_call(
    _final_body,
    grid=(GRID,),
    in_specs=[_row_spec, _row_spec, _row_spec],
    out_specs=_row_spec,
    out_shape=jax.ShapeDtypeStruct((N, D), jnp.float32),
)


_mesh = plsc.VectorSubcoreMesh(
    core_axis_name="c", subcore_axis_name="s", num_cores=NC, num_subcores=NS)


@functools.partial(
    pl.kernel,
    out_type=jax.ShapeDtypeStruct((NC, NPAD, D), jnp.float32),
    mesh=_mesh,
    scratch_types=[
        pltpu.VMEM((C,), jnp.int32),
        pltpu.VMEM((C,), jnp.int32),
        pltpu.VMEM((C, D), jnp.float32),
        pltpu.VMEM((C, D), jnp.float32),
        pltpu.VMEM((C, D), jnp.float32),
        pltpu.VMEM((RZ, D), jnp.float32),
        pltpu.VMEM_SHARED((NPAD, D), jnp.float32),
        pltpu.SemaphoreType.DMA,
    ],
)
def _edge_kernel(kn_hbm, qn_hbm, v_hbm, src_hbm, dst_hbm, out_hbm,
                 srcv, dstv, kdv, qsv, vsv, zbuf, agg, sem):
    c = lax.axis_index("c")
    s = lax.axis_index("s")
    wid = s * NC + c

    def _zero_row(r, carry):
        for j in range(D // 16):
            zbuf[r, pl.ds(j * 16, 16)] = jnp.zeros((16,), jnp.float32)
        return carry
    lax.fori_loop(0, RZ, _zero_row, 0)
    for t in range(RPT // RZ):
        pltpu.sync_copy(zbuf, agg.at[pl.ds(s * RPT + t * RZ, RZ)])
    plsc.subcore_barrier()

    def _chunk(nc_i, carry):
        base = pl.multiple_of(wid * EPW + nc_i * C, 8)
        pltpu.sync_copy(src_hbm.at[pl.ds(base, C)], srcv)
        pltpu.sync_copy(dst_hbm.at[pl.ds(base, C)], dstv)
        d1 = pltpu.async_copy(kn_hbm.at[dstv], kdv, sem)
        d2 = pltpu.async_copy(qn_hbm.at[srcv], qsv, sem)
        d3 = pltpu.async_copy(v_hbm.at[srcv], vsv, sem)
        d1.wait()
        d2.wait()
        d3.wait()

        def _row(r, rc):
            for j in range(D // 16):
                sl = pl.ds(j * 16, 16)
                z = kdv[r, sl] + qsv[r, sl]
                kdv[r, sl] = vsv[r, sl] / (1.0 + jnp.exp(z))
            return rc
        lax.fori_loop(0, C, _row, 0)

        pltpu.sync_copy(kdv, agg.at[dstv], add=True)
        return carry
    lax.fori_loop(0, NCHUNK, _chunk, 0)

    plsc.subcore_barrier()
    for t in range(RPT // RZ):
        off = s * RPT + t * RZ
        pltpu.sync_copy(agg.at[pl.ds(off, RZ)], zbuf)
        pltpu.sync_copy(zbuf, out_hbm.at[c, pl.ds(off, RZ)])


def kernel(x, edge_index, Wk1, bk1, Wq1, bq1, Wv1, bv1, Ws1, b1,
           Wk2, bk2, Wq2, bq2, Wv2, bv2, Ws2, b2):
    src = edge_index[0]
    dst = edge_index[1]

    w1 = jnp.concatenate([-Wk1, -Wq1, Wv1, Ws1], axis=1)
    b1c = jnp.broadcast_to(
        jnp.concatenate([-bk1, -bq1, bv1, b1])[None, :], (8, 4 * D))
    w2 = jnp.concatenate([-Wk2, -Wq2, Wv2, Ws2], axis=1)
    b2c = jnp.broadcast_to(
        jnp.concatenate([-bk2, -bq2, bv2, b2])[None, :], (8, 4 * D))

    kn1, qn1, v1, s1 = _proj(x, w1, b1c)
    part1 = _edge_kernel(kn1, qn1, v1, src, dst)
    kn2, qn2, v2, s2 = _relu_proj(part1[0, :N], part1[1, :N], s1, w2, b2c)
    part2 = _edge_kernel(kn2, qn2, v2, src, dst)
    return _final(part2[0, :N], part2[1, :N], s2)


# speedup vs baseline: 8.5910x; 1.7869x over previous
import functools

import jax
import jax.numpy as jnp
from jax import lax
from jax.experimental import pallas as pl
from jax.experimental.pallas import tpu as pltpu
from jax.experimental.pallas import tpu_sc as plsc

N = 10000
E = 320000
D = 128

NC = 2
NS = 16
NW = NC * NS
EPW = E // NW
C = 40
NCHUNK = EPW // C
G = 10
SG = NCHUNK // G
NPAD = 10240
RPT = NPAD // NS

MBLK = 1000
GRID = N // MBLK


def _proj_body(x_ref, w_ref, b_ref, kn_ref, qn_ref, v_ref, s_ref):
    y = jnp.dot(x_ref[...], w_ref[...], preferred_element_type=jnp.float32)
    y = y + b_ref[0:1, :]
    kn_ref[...] = y[:, 0 * D:1 * D]
    qn_ref[...] = y[:, 1 * D:2 * D]
    v_ref[...] = y[:, 2 * D:3 * D]
    s_ref[...] = y[:, 3 * D:4 * D]


def _relu_proj_body(p0_ref, p1_ref, s1_ref, w_ref, b_ref,
                    kn_ref, qn_ref, v_ref, s_ref):
    h = jnp.maximum(p0_ref[...] + p1_ref[...] + s1_ref[...], 0.0)
    y = jnp.dot(h, w_ref[...], preferred_element_type=jnp.float32)
    y = y + b_ref[0:1, :]
    kn_ref[...] = y[:, 0 * D:1 * D]
    qn_ref[...] = y[:, 1 * D:2 * D]
    v_ref[...] = y[:, 2 * D:3 * D]
    s_ref[...] = y[:, 3 * D:4 * D]


def _final_body(p0_ref, p1_ref, s2_ref, o_ref):
    o_ref[...] = p0_ref[...] + p1_ref[...] + s2_ref[...]


_row_spec = pl.BlockSpec((MBLK, D), lambda i: (i, 0))
_w_spec = pl.BlockSpec((D, 4 * D), lambda i: (0, 0))
_b_spec = pl.BlockSpec((8, 4 * D), lambda i: (0, 0))
_out4 = tuple(jax.ShapeDtypeStruct((N, D), jnp.float32) for _ in range(4))

_proj = pl.pallas_call(
    _proj_body,
    grid=(GRID,),
    in_specs=[_row_spec, _w_spec, _b_spec],
    out_specs=(_row_spec,) * 4,
    out_shape=_out4,
)

_relu_proj = pl.pallas_call(
    _relu_proj_body,
    grid=(GRID,),
    in_specs=[_row_spec, _row_spec, _row_spec, _w_spec, _b_spec],
    out_specs=(_row_spec,) * 4,
    out_shape=_out4,
)

_final = pl.pallas_call(
    _final_body,
    grid=(GRID,),
    in_specs=[_row_spec, _row_spec, _row_spec],
    out_specs=_row_spec,
    out_shape=jax.ShapeDtypeStruct((N, D), jnp.float32),
)


_mesh = plsc.VectorSubcoreMesh(
    core_axis_name="c", subcore_axis_name="s", num_cores=NC, num_subcores=NS)


@functools.partial(
    pl.kernel,
    out_type=jax.ShapeDtypeStruct((NC, NPAD, D), jnp.float32),
    mesh=_mesh,
    scratch_types=[
        [pltpu.VMEM((G, 2, C), jnp.int32)] * 2,
        [pltpu.VMEM((C, D), jnp.float32)] * 2,
        [pltpu.VMEM((C, D), jnp.float32)] * 2,
        [pltpu.VMEM((C, D), jnp.float32)] * 2,
        [pltpu.VMEM((C, D), jnp.float32)] * 2,
        pltpu.VMEM_SHARED((NPAD, D), jnp.float32),
        [pltpu.SemaphoreType.DMA] * 2,
        [pltpu.SemaphoreType.DMA] * 2,
        pltpu.SemaphoreType.DMA,
    ],
)
def _edge_kernel(kn_hbm, qn_hbm, v_hbm, sd_hbm, out_hbm,
                 idxv, kdv, qsv, vsv, mv, agg, gsem, ssem, isem):
    c = lax.axis_index("c")
    s = lax.axis_index("s")
    wid = s * NC + c

    def _zero_row(r, carry):
        for j in range(D // 16):
            kdv[0][r, pl.ds(j * 16, 16)] = jnp.zeros((16,), jnp.float32)
        return carry
    lax.fori_loop(0, C, _zero_row, 0)
    for t in range(RPT // C):
        pltpu.async_copy(kdv[0], agg.at[pl.ds(s * RPT + t * C, C)], gsem[0])
    for t in range(RPT // C):
        pltpu.make_async_copy(kdv[0], agg.at[pl.ds(s * RPT, C)], gsem[0]).wait()
    plsc.subcore_barrier()

    def _gather(g, b, sb):
        pltpu.async_copy(kn_hbm.at[idxv[sb].at[g, 1]], kdv[b], gsem[b])
        pltpu.async_copy(qn_hbm.at[idxv[sb].at[g, 0]], qsv[b], gsem[b])
        pltpu.async_copy(v_hbm.at[idxv[sb].at[g, 0]], vsv[b], gsem[b])

    def _wait_gather(b):
        i0 = idxv[0].at[0, 0]
        pltpu.make_async_copy(kn_hbm.at[i0], kdv[b], gsem[b]).wait()
        pltpu.make_async_copy(qn_hbm.at[i0], qsv[b], gsem[b]).wait()
        pltpu.make_async_copy(v_hbm.at[i0], vsv[b], gsem[b]).wait()

    def _wait_scatter(b):
        pltpu.make_async_copy(mv[b], agg.at[idxv[0].at[0, 1]], ssem[b]).wait()

    def _step(g, b, sb, wait_scat, gather_next):
        if gather_next:
            _gather(g + 1, 1 - b, sb)
        _wait_gather(b)
        if wait_scat:
            _wait_scatter(b)

        def _row(r, rc):
            for j in range(D // 16):
                sl = pl.ds(j * 16, 16)
                z = kdv[b][r, sl] + qsv[b][r, sl]
                mv[b][r, sl] = vsv[b][r, sl] / (1.0 + jnp.exp(z))
            return rc
        lax.fori_loop(0, C, _row, 0)
        pltpu.async_copy(mv[b], agg.at[idxv[sb].at[g, 1]], ssem[b], add=True)

    def _super(sc, sb, first):
        if not first:
            pltpu.make_async_copy(sd_hbm.at[wid, 0], idxv[sb], isem).wait()
            _wait_scatter(0)
            _wait_scatter(1)
        pltpu.async_copy(sd_hbm.at[wid, sc + 1], idxv[1 - sb], isem)
        _gather(0, 0, sb)
        _step(0, 0, sb, False, True)
        _step(1, 1, sb, False, True)

        def _pairs(p, cc):
            g = 2 * p
            _step(g, 0, sb, True, True)
            _step(g + 1, 1, sb, True, True)
            return cc
        lax.fori_loop(1, G // 2 - 1, _pairs, 0)
        _step(G - 2, 0, sb, True, True)
        _step(G - 1, 1, sb, True, False)

    pltpu.sync_copy(sd_hbm.at[wid, 0], idxv[0])
    _super(0, 0, True)

    def _souter(p, cc):
        _super(2 * p + 1, 1, False)
        _super(2 * p + 2, 0, False)
        return cc
    lax.fori_loop(0, (SG - 1) // 2, _souter, 0)
    pltpu.make_async_copy(sd_hbm.at[wid, 0], idxv[1], isem).wait()
    _wait_scatter(0)
    _wait_scatter(1)

    plsc.subcore_barrier()
    pltpu.sync_copy(agg.at[pl.ds(s * RPT, RPT)],
                    out_hbm.at[c, pl.ds(s * RPT, RPT)])


def kernel(x, edge_index, Wk1, bk1, Wq1, bq1, Wv1, bv1, Ws1, b1,
           Wk2, bk2, Wq2, bq2, Wv2, bv2, Ws2, b2):
    src = edge_index[0].reshape(NW, SG, G, 1, C)
    dst = edge_index[1].reshape(NW, SG, G, 1, C)
    sd = jnp.concatenate([src, dst], axis=3)
    sd = jnp.concatenate(
        [sd, jnp.zeros((NW, 1, G, 2, C), jnp.int32)], axis=1)

    w1 = jnp.concatenate([-Wk1, -Wq1, Wv1, Ws1], axis=1)
    b1c = jnp.broadcast_to(
        jnp.concatenate([-bk1, -bq1, bv1, b1])[None, :], (8, 4 * D))
    w2 = jnp.concatenate([-Wk2, -Wq2, Wv2, Ws2], axis=1)
    b2c = jnp.broadcast_to(
        jnp.concatenate([-bk2, -bq2, bv2, b2])[None, :], (8, 4 * D))

    kn1, qn1, v1, s1 = _proj(x, w1, b1c)
    part1 = _edge_kernel(kn1, qn1, v1, sd)
    kn2, qn2, v2, s2 = _relu_proj(part1[0, :N], part1[1, :N], s1, w2, b2c)
    part2 = _edge_kernel(kn2, qn2, v2, sd)
    return _final(part2[0, :N], part2[1, :N], s2)


# speedup vs baseline: 8.6439x; 1.0062x over previous
import functools

import jax
import jax.numpy as jnp
from jax import lax
from jax.experimental import pallas as pl
from jax.experimental.pallas import tpu as pltpu
from jax.experimental.pallas import tpu_sc as plsc

N = 10000
E = 320000
D = 128

NC = 2
NS = 16
NW = NC * NS
EPW = E // NW
C = 40
NCHUNK = EPW // C
G = 10
SG = NCHUNK // G
NPAD = 10240
RPT = NPAD // NS

MBLK = 1000
GRID = N // MBLK


def _proj_body(x_ref, w_ref, b_ref, kn_ref, qn_ref, v_ref, s_ref):
    y = jnp.dot(x_ref[...], w_ref[...], preferred_element_type=jnp.float32)
    y = y + b_ref[0:1, :]
    kn_ref[...] = y[:, 0 * D:1 * D]
    qn_ref[...] = y[:, 1 * D:2 * D]
    v_ref[...] = y[:, 2 * D:3 * D]
    s_ref[...] = y[:, 3 * D:4 * D]


def _relu_proj_body(p0_ref, p1_ref, s1_ref, w_ref, b_ref,
                    kn_ref, qn_ref, v_ref, s_ref):
    h = jnp.maximum(p0_ref[...] + p1_ref[...] + s1_ref[...], 0.0)
    y = jnp.dot(h, w_ref[...], preferred_element_type=jnp.float32)
    y = y + b_ref[0:1, :]
    kn_ref[...] = y[:, 0 * D:1 * D]
    qn_ref[...] = y[:, 1 * D:2 * D]
    v_ref[...] = y[:, 2 * D:3 * D]
    s_ref[...] = y[:, 3 * D:4 * D]


def _final_body(p0_ref, p1_ref, s2_ref, o_ref):
    o_ref[...] = p0_ref[...] + p1_ref[...] + s2_ref[...]


_row_spec = pl.BlockSpec((MBLK, D), lambda i: (i, 0))
_w_spec = pl.BlockSpec((D, 4 * D), lambda i: (0, 0))
_b_spec = pl.BlockSpec((8, 4 * D), lambda i: (0, 0))
_out4 = tuple(jax.ShapeDtypeStruct((N, D), jnp.float32) for _ in range(4))

_proj = pl.pallas_call(
    _proj_body,
    grid=(GRID,),
    in_specs=[_row_spec, _w_spec, _b_spec],
    out_specs=(_row_spec,) * 4,
    out_shape=_out4,
)

_relu_proj = pl.pallas_call(
    _relu_proj_body,
    grid=(GRID,),
    in_specs=[_row_spec, _row_spec, _row_spec, _w_spec, _b_spec],
    out_specs=(_row_spec,) * 4,
    out_shape=_out4,
)

_final = pl.pallas_call(
    _final_body,
    grid=(GRID,),
    in_specs=[_row_spec, _row_spec, _row_spec],
    out_specs=_row_spec,
    out_shape=jax.ShapeDtypeStruct((N, D), jnp.float32),
)


_mesh = plsc.VectorSubcoreMesh(
    core_axis_name="c", subcore_axis_name="s", num_cores=NC, num_subcores=NS)


@functools.partial(
    pl.kernel,
    out_type=jax.ShapeDtypeStruct((NC, NPAD, D), jnp.float32),
    mesh=_mesh,
    scratch_types=[
        [pltpu.VMEM((G, 2, C), jnp.int32)] * 2,
        [pltpu.VMEM((C, D), jnp.float32)] * 2,
        [pltpu.VMEM((C, D), jnp.float32)] * 2,
        [pltpu.VMEM((C, D), jnp.float32)] * 2,
        [pltpu.VMEM((C, D), jnp.float32)] * 2,
        pltpu.VMEM_SHARED((NPAD, D), jnp.float32),
        [pltpu.SemaphoreType.DMA] * 2,
        [pltpu.SemaphoreType.DMA] * 2,
        pltpu.SemaphoreType.DMA,
    ],
)
def _edge_kernel(kn_hbm, qn_hbm, v_hbm, sd_hbm, out_hbm,
                 idxv, kdv, qsv, vsv, mv, agg, gsem, ssem, isem):
    c = lax.axis_index("c")
    s = lax.axis_index("s")
    wid = s * NC + c

    def _zero_row(r, carry):
        for j in range(D // 16):
            kdv[0][r, pl.ds(j * 16, 16)] = jnp.zeros((16,), jnp.float32)
        return carry
    lax.fori_loop(0, C, _zero_row, 0)
    for t in range(RPT // C):
        pltpu.async_copy(kdv[0], agg.at[pl.ds(s * RPT + t * C, C)], gsem[0])
    for t in range(RPT // C):
        pltpu.make_async_copy(kdv[0], agg.at[pl.ds(s * RPT, C)], gsem[0]).wait()
    plsc.subcore_barrier()

    def _gather(g, b, sb):
        pltpu.async_copy(kn_hbm.at[idxv[sb].at[g, 1]], kdv[b], gsem[b])
        pltpu.async_copy(qn_hbm.at[idxv[sb].at[g, 0]], qsv[b], gsem[b])
        pltpu.async_copy(v_hbm.at[idxv[sb].at[g, 0]], vsv[b], gsem[b])

    def _wait_gather(b):
        i0 = idxv[0].at[0, 0]
        pltpu.make_async_copy(kn_hbm.at[i0], kdv[b], gsem[b]).wait()
        pltpu.make_async_copy(qn_hbm.at[i0], qsv[b], gsem[b]).wait()
        pltpu.make_async_copy(v_hbm.at[i0], vsv[b], gsem[b]).wait()

    def _wait_scatter(b):
        pltpu.make_async_copy(mv[b], agg.at[idxv[0].at[0, 1]], ssem[b]).wait()

    def _step(g, b, sb, wait_scat, gather_next):
        if gather_next:
            _gather(g + 1, 1 - b, sb)
        _wait_gather(b)
        if wait_scat:
            _wait_scatter(b)

        @plsc.parallel_loop(0, C, 1, unroll=1)
        def _row(r):
            for j in range(D // 16):
                sl = pl.ds(j * 16, 16)
                z = kdv[b][r, sl] + qsv[b][r, sl]
                mv[b][r, sl] = vsv[b][r, sl] / (1.0 + jnp.exp(z))
        pltpu.async_copy(mv[b], agg.at[idxv[sb].at[g, 1]], ssem[b], add=True)

    def _super(sc, sb, first):
        if not first:
            pltpu.make_async_copy(sd_hbm.at[wid, 0], idxv[sb], isem).wait()
            _wait_scatter(0)
            _wait_scatter(1)
        pltpu.async_copy(sd_hbm.at[wid, sc + 1], idxv[1 - sb], isem)
        _gather(0, 0, sb)
        _step(0, 0, sb, False, True)
        _step(1, 1, sb, False, True)

        def _pairs(p, cc):
            g = 2 * p
            _step(g, 0, sb, True, True)
            _step(g + 1, 1, sb, True, True)
            return cc
        lax.fori_loop(1, G // 2 - 1, _pairs, 0)
        _step(G - 2, 0, sb, True, True)
        _step(G - 1, 1, sb, True, False)

    pltpu.sync_copy(sd_hbm.at[wid, 0], idxv[0])
    _super(0, 0, True)

    def _souter(p, cc):
        _super(2 * p + 1, 1, False)
        _super(2 * p + 2, 0, False)
        return cc
    lax.fori_loop(0, (SG - 1) // 2, _souter, 0)
    pltpu.make_async_copy(sd_hbm.at[wid, 0], idxv[1], isem).wait()
    _wait_scatter(0)
    _wait_scatter(1)

    plsc.subcore_barrier()
    pltpu.sync_copy(agg.at[pl.ds(s * RPT, RPT)],
                    out_hbm.at[c, pl.ds(s * RPT, RPT)])


def kernel(x, edge_index, Wk1, bk1, Wq1, bq1, Wv1, bv1, Ws1, b1,
           Wk2, bk2, Wq2, bq2, Wv2, bv2, Ws2, b2):
    src = edge_index[0].reshape(NW, SG, G, 1, C)
    dst = edge_index[1].reshape(NW, SG, G, 1, C)
    sd = jnp.concatenate([src, dst], axis=3)
    sd = jnp.concatenate(
        [sd, jnp.zeros((NW, 1, G, 2, C), jnp.int32)], axis=1)

    w1 = jnp.concatenate([-Wk1, -Wq1, Wv1, Ws1], axis=1)
    b1c = jnp.broadcast_to(
        jnp.concatenate([-bk1, -bq1, bv1, b1])[None, :], (8, 4 * D))
    w2 = jnp.concatenate([-Wk2, -Wq2, Wv2, Ws2], axis=1)
    b2c = jnp.broadcast_to(
        jnp.concatenate([-bk2, -bq2, bv2, b2])[None, :], (8, 4 * D))

    kn1, qn1, v1, s1 = _proj(x, w1, b1c)
    part1 = _edge_kernel(kn1, qn1, v1, sd)
    kn2, qn2, v2, s2 = _relu_proj(part1[0, :N], part1[1, :N], s1, w2, b2c)
    part2 = _edge_kernel(kn2, qn2, v2, sd)
    return _final(part2[0, :N], part2[1, :N], s2)
